# Initial kernel scaffold; baseline (speedup 1.0000x reference)
import jax
import jax.numpy as jnp
from jax.experimental import pallas as pl


def kernel(x, edge_index, Wg, al, ar, W1, b1, W2, Wp, bp, Wq, Wk, Wv, Wfc, bfc, Wres, bres, res_w, ln_g, ln_b):
    raise NotImplementedError("write your pallas kernel here")


# speedup vs baseline: 4.5603x; 4.5603x over previous
import functools
import math

import jax
import jax.numpy as jnp
import numpy as np
from jax import lax
from jax.experimental import pallas as pl
from jax.experimental.pallas import tpu as pltpu
from jax.experimental.pallas import tpu_sc as plsc

T, R, N, E, D, H = 4, 2, 10000, 160000, 256, 256
G = T * R
BTOT = G * E
NC, NS = 2, 16
NW = NC * NS
EW = E // NW
CB = 40
CH = EW // CB
GK = 160
GSTEPS = BTOT // NW // GK
NB = 25
NBS = N // NB


def _make_pe():
    pe = np.zeros((T, H), dtype=np.float32)
    for i in range(T):
        for k in range(0, H, 2):
            div_term = math.exp(k * -math.log(100000.0) / H)
            pe[i][k] = math.sin((i + 1) * div_term)
            if k + 1 < H:
                pe[i][k + 1] = math.cos((i + 1) * div_term)
    return pe


_PE = _make_pe()


def _pre_body(x_ref, wg_ref, ar_ref, h_ref, er1_ref):
    h = jnp.dot(x_ref[0], wg_ref[0, 0], preferred_element_type=jnp.float32)
    h_ref[0] = h
    er = jnp.dot(h, ar_ref[pl.program_id(0)][:, None],
                 preferred_element_type=jnp.float32)
    er1_ref[0] = h[:, :128] * 0.0 + er


def _pre_call(x, Wg, ar8):
    return pl.pallas_call(
        _pre_body,
        grid=(G, NB),
        in_specs=[
            pl.BlockSpec((1, NBS, D), lambda g, i: (g // R, i, 0)),
            pl.BlockSpec((1, 1, D, H), lambda g, i: (g // R, g % R, 0, 0)),
            pl.BlockSpec((G, H), lambda g, i: (0, 0)),
        ],
        out_specs=[
            pl.BlockSpec((1, NBS, H), lambda g, i: (g, i, 0)),
            pl.BlockSpec((1, NBS, 128), lambda g, i: (g, i, 0)),
        ],
        out_shape=[
            jax.ShapeDtypeStruct((G, N, H), jnp.float32),
            jax.ShapeDtypeStruct((G, N, 128), jnp.float32),
        ],
    )(x, Wg, ar8)


def _idx_body(ei_ref, src_ref, dst_ref):
    g = pl.program_id(0)
    base = g * N
    ei = ei_ref[0, 0]
    src_ref[0, 0] = ei[0] + base
    dst_ref[0, 0] = ei[1] + base


def _idx_call(edge_index):
    return pl.pallas_call(
        _idx_body,
        grid=(G,),
        in_specs=[pl.BlockSpec((1, 1, 2, E), lambda g: (g // R, g % R, 0, 0))],
        out_specs=[
            pl.BlockSpec((1, 1, E), lambda g: (g, 0, 0)),
            pl.BlockSpec((1, 1, E), lambda g: (g, 0, 0)),
        ],
        out_shape=[
            jax.ShapeDtypeStruct((G, 1, E), jnp.int32),
            jax.ShapeDtypeStruct((G, 1, E), jnp.int32),
        ],
    )(edge_index)


def _make_gather(dt):
    mesh = plsc.VectorSubcoreMesh(core_axis_name="c", subcore_axis_name="s")
    b_per_w = BTOT // NW

    @functools.partial(
        pl.kernel,
        mesh=mesh,
        out_type=jax.ShapeDtypeStruct((BTOT, dt), jnp.float32),
        scratch_types=[
            pltpu.VMEM((GK,), jnp.int32),
            pltpu.VMEM((GK, dt), jnp.float32),
            pltpu.SemaphoreType.DMA,
        ],
    )
    def gath(table_hbm, idx_hbm, out_hbm, idx_v, rows_v, sem):
        wid = lax.axis_index("s") * NC + lax.axis_index("c")
        base = wid * b_per_w

        def body(j, carry):
            off = base + j * GK
            pltpu.sync_copy(idx_hbm.at[pl.ds(off, GK)], idx_v)
            pltpu.async_copy(table_hbm.at[idx_v], rows_v, sem).wait()
            pltpu.sync_copy(rows_v, out_hbm.at[pl.ds(off, GK)])
            return carry

        lax.fori_loop(0, GSTEPS, body, 0)

    return gath


_gather_h = _make_gather(H)
_gather_e = _make_gather(128)


EB = 640
EBLKS = E // EB


def _edge_body(hs_ref, erd_ref, al_ref, wh0_ref, wh1_ref, whz_ref):
    hs = hs_ref[...]
    el = jnp.dot(hs, al_ref[pl.program_id(0)][:, None],
                 preferred_element_type=jnp.float32)
    e = el + erd_ref[:, 0:1]
    e = jnp.where(e > 0, e, 0.2 * e)
    w = jnp.exp(e)
    wh = hs * w
    wh0_ref[...] = wh[:, :128]
    wh1_ref[...] = wh[:, 128:]
    whz_ref[...] = hs[:, :128] * 0.0 + w


def _edge_call(hs, erd, al8):
    return pl.pallas_call(
        _edge_body,
        grid=(G, EBLKS),
        in_specs=[
            pl.BlockSpec((EB, H), lambda g, i: (g * EBLKS + i, 0)),
            pl.BlockSpec((EB, 128), lambda g, i: (g * EBLKS + i, 0)),
            pl.BlockSpec((G, H), lambda g, i: (0, 0)),
        ],
        out_specs=[
            pl.BlockSpec((EB, 128), lambda g, i: (g * EBLKS + i, 0)),
            pl.BlockSpec((EB, 128), lambda g, i: (g * EBLKS + i, 0)),
            pl.BlockSpec((EB, 128), lambda g, i: (g * EBLKS + i, 0)),
        ],
        out_shape=[
            jax.ShapeDtypeStruct((BTOT, 128), jnp.float32),
            jax.ShapeDtypeStruct((BTOT, 128), jnp.float32),
            jax.ShapeDtypeStruct((BTOT, 128), jnp.float32),
        ],
    )(hs, erd, al8)


_scatter_mesh = plsc.VectorSubcoreMesh(core_axis_name="c", subcore_axis_name="s")


@functools.partial(
    pl.kernel,
    mesh=_scatter_mesh,
    out_type=jax.ShapeDtypeStruct((G, NC, N, 128), jnp.float32),
    scratch_types=[
        pltpu.VMEM((CH, CB), jnp.int32),
        pltpu.VMEM((CB, 128), jnp.float32),
        pltpu.VMEM_SHARED((N, 128), jnp.float32),
    ],
)
def _scatter(wh_hbm, dst_hbm, z128_hbm, num_hbm, dst_v, rows_v, shn):
    cid = lax.axis_index("c")
    sid = lax.axis_index("s")
    wid = sid * NC + cid

    def graph_body(g, carry):
        @pl.when(sid == 0)
        def _():
            pltpu.sync_copy(z128_hbm, shn)

        plsc.subcore_barrier()
        pltpu.sync_copy(dst_hbm.at[g, wid], dst_v)
        ebase = g * E + wid * EW

        def chunk(ch, c2):
            off = ebase + ch * CB
            pltpu.sync_copy(wh_hbm.at[pl.ds(off, CB)], rows_v)
            pltpu.sync_copy(rows_v, shn.at[dst_v.at[ch]], add=True)
            return c2

        lax.fori_loop(0, CH, chunk, 0)
        plsc.subcore_barrier()

        @pl.when(sid == 0)
        def _():
            pltpu.sync_copy(shn, num_hbm.at[g, cid])

        plsc.subcore_barrier()
        return carry

    lax.fori_loop(0, G, graph_body, 0)


def _hrel_block(n0, n1, z, t, r):
    g = t * R + r
    num = jnp.concatenate([n0[g, 0] + n0[g, 1], n1[g, 0] + n1[g, 1]], axis=-1)
    zz = z[g, 0, :, 0:1] + z[g, 1, :, 0:1]
    return num / (zz + 1e-9)


def _e1_body(n0_ref, n1_ref, z_ref, w1_ref, b1_ref, w2b_ref, part_ref):
    n0 = n0_ref[...]
    n1 = n1_ref[...]
    z = z_ref[...]
    rows = []
    for t in range(T):
        vals = []
        for r in range(R):
            hr = _hrel_block(n0, n1, z, t, r)
            s = jnp.dot(jnp.tanh(jnp.dot(hr, w1_ref[t], preferred_element_type=jnp.float32)
                                 + b1_ref[t][None, :]),
                        w2b_ref[t], preferred_element_type=jnp.float32)
            vals.append(jnp.sum(s, axis=0, keepdims=True))
        rows.append(jnp.concatenate(vals, axis=0)[None])
    part_ref[0] = jnp.concatenate(rows, axis=0)


def _e1_call(num0, num1, z, W1, b1, W2b):
    return pl.pallas_call(
        _e1_body,
        grid=(NB,),
        in_specs=[
            pl.BlockSpec((G, NC, NBS, 128), lambda i: (0, 0, i, 0)),
            pl.BlockSpec((G, NC, NBS, 128), lambda i: (0, 0, i, 0)),
            pl.BlockSpec((G, NC, NBS, 128), lambda i: (0, 0, i, 0)),
            pl.BlockSpec((T, H, H), lambda i: (0, 0, 0)),
            pl.BlockSpec((T, H), lambda i: (0, 0)),
            pl.BlockSpec((T, H, 128), lambda i: (0, 0, 0)),
        ],
        out_specs=pl.BlockSpec((1, T, R, 128), lambda i: (i, 0, 0, 0)),
        out_shape=jax.ShapeDtypeStruct((NB, T, R, 128), jnp.float32),
    )(num0, num1, z, W1, b1, W2b)


def _e2_body(n0_ref, n1_ref, z_ref, part_ref, x_ref,
             wp_ref, bp_ref, wq_ref, wk_ref, wv_ref, wfc_ref, bfc_ref,
             wres_ref, bres_ref, resb_ref, lng_ref, lnb_ref, pe_ref, out_ref):
    n0 = n0_ref[...]
    n1 = n1_ref[...]
    z = z_ref[...]
    w2 = jnp.sum(part_ref[...], axis=0) / N
    m = jnp.max(w2, axis=1, keepdims=True)
    bexp = jnp.exp(w2 - m)
    beta = bexp / jnp.sum(bexp, axis=1, keepdims=True)

    qs, ks, vs = [], [], []
    for t in range(T):
        b0 = jnp.concatenate([beta[t, 0:1], beta[t, 0:1]], axis=1)
        b1 = jnp.concatenate([beta[t, 1:2], beta[t, 1:2]], axis=1)
        te = b0 * _hrel_block(n0, n1, z, t, 0) \
            + b1 * _hrel_block(n0, n1, z, t, 1)
        h2 = jnp.dot(te, wp_ref[...], preferred_element_type=jnp.float32) \
            + bp_ref[0][None, :] + pe_ref[t][None, :]
        qs.append(jnp.dot(h2, wq_ref[...], preferred_element_type=jnp.float32))
        ks.append(jnp.dot(h2, wk_ref[...], preferred_element_type=jnp.float32))
        vs.append(jnp.dot(h2, wv_ref[...], preferred_element_type=jnp.float32))

    alpha = 1.0 / (1.0 + jnp.exp(-resb_ref[0, 0]))
    for t in range(T):
        qk = jnp.concatenate(
            [jnp.sum(qs[t] * ks[s], axis=1, keepdims=True) for s in range(T)],
            axis=1)
        mm = jnp.max(qk, axis=1, keepdims=True)
        ee = jnp.exp(qk - mm)
        a = ee / jnp.sum(ee, axis=1, keepdims=True)
        o = sum(a[:, s:s + 1] * vs[s] for s in range(T))
        f = jnp.maximum(
            jnp.dot(o, wfc_ref[...], preferred_element_type=jnp.float32)
            + bfc_ref[0][None, :], 0.0)
        res = jnp.dot(x_ref[t], wres_ref[...], preferred_element_type=jnp.float32) \
            + bres_ref[0][None, :]
        ot = f * alpha + res * (1.0 - alpha)
        mu = jnp.mean(ot, axis=1, keepdims=True)
        var = jnp.mean((ot - mu) * (ot - mu), axis=1, keepdims=True)
        out_ref[t] = (ot - mu) / jnp.sqrt(var + 1e-5) * lng_ref[0][None, :] \
            + lnb_ref[0][None, :]


def _e2_call(num0, num1, z, part, x, Wp, bp_r, Wq, Wk, Wv, Wfc, bfc_r,
             Wres, bres_r, res_b, lng_r, lnb_r, pe):
    full2 = lambda i: (0, 0)
    return pl.pallas_call(
        _e2_body,
        grid=(NB,),
        in_specs=[
            pl.BlockSpec((G, NC, NBS, 128), lambda i: (0, 0, i, 0)),
            pl.BlockSpec((G, NC, NBS, 128), lambda i: (0, 0, i, 0)),
            pl.BlockSpec((G, NC, NBS, 128), lambda i: (0, 0, i, 0)),
            pl.BlockSpec((NB, T, R, 128), lambda i: (0, 0, 0, 0)),
            pl.BlockSpec((T, NBS, D), lambda i: (0, i, 0)),
            pl.BlockSpec((H, H), full2),
            pl.BlockSpec((1, H), full2),
            pl.BlockSpec((H, H), full2),
            pl.BlockSpec((H, H), full2),
            pl.BlockSpec((H, H), full2),
            pl.BlockSpec((H, H), full2),
            pl.BlockSpec((1, H), full2),
            pl.BlockSpec((D, H), full2),
            pl.BlockSpec((1, H), full2),
            pl.BlockSpec((1, 128), full2),
            pl.BlockSpec((1, H), full2),
            pl.BlockSpec((1, H), full2),
            pl.BlockSpec((T, H), full2),
        ],
        out_specs=pl.BlockSpec((T, NBS, H), lambda i: (0, i, 0)),
        out_shape=jax.ShapeDtypeStruct((T, N, H), jnp.float32),
    )(num0, num1, z, part, x, Wp, bp_r, Wq, Wk, Wv, Wfc, bfc_r,
      Wres, bres_r, res_b, lng_r, lnb_r, pe)


def kernel(x, edge_index, Wg, al, ar, W1, b1, W2, Wp, bp, Wq, Wk, Wv,
           Wfc, bfc, Wres, bres, res_w, ln_g, ln_b):
    h8, er1 = _pre_call(x, Wg, ar.reshape(G, H))
    srcg, dstg = _idx_call(edge_index)
    hs = _gather_h(h8.reshape(G * N, H), srcg.reshape(BTOT))
    erd = _gather_e(er1.reshape(G * N, 128), dstg.reshape(BTOT))
    wh0, wh1, whz = _edge_call(hs, erd, al.reshape(G, H))
    dst2d = edge_index[:, :, 1].reshape(G, NW, CH, CB)
    z128 = jnp.zeros((N, 128), jnp.float32)
    num0 = _scatter(wh0, dst2d, z128)
    num1 = _scatter(wh1, dst2d, z128)
    numz = _scatter(whz, dst2d, z128)
    part = _e1_call(num0, num1, numz, W1, b1,
                    jnp.broadcast_to(W2[..., None], (T, H, 128)))
    return _e2_call(num0, num1, numz, part, x,
                    Wp, bp.reshape(1, H), Wq, Wk, Wv, Wfc, bfc.reshape(1, H),
                    Wres, bres.reshape(1, H),
                    jnp.broadcast_to(res_w.reshape(1, 1), (1, 128)),
                    ln_g.reshape(1, H), ln_b.reshape(1, H), jnp.asarray(_PE))
